# Initial kernel scaffold; baseline (speedup 1.0000x reference)
import jax
import jax.numpy as jnp
from jax.experimental import pallas as pl


def kernel(x, edge_index, W1, b1, Wc1, bc1, Wc2, bc2, W2, b2):
    raise NotImplementedError("write your pallas kernel here")


# speedup vs baseline: 26.8876x; 26.8876x over previous
import functools

import jax
import jax.numpy as jnp
from jax import lax
from jax.experimental import pallas as pl
from jax.experimental.pallas import tpu as pltpu
from jax.experimental.pallas import tpu_sc as plsc

NC, NS = 2, 16
NW = NC * NS
CHUNK = 2000


def _sc_mesh():
    return plsc.VectorSubcoreMesh(core_axis_name="c", subcore_axis_name="s",
                                  num_cores=NC, num_subcores=NS)


_SC_PARAMS = pltpu.CompilerParams(use_tc_tiling_on_sc=False)


def _make_degree_kernel(E, N):
    per_tile = E // NW
    nchunk = per_tile // CHUNK
    assert per_tile % CHUNK == 0 and E % NW == 0
    nsplit = 10
    rows_per = N // nsplit
    assert N % nsplit == 0 and rows_per % 8 == 0

    @functools.partial(
        pl.kernel,
        out_type=(jax.ShapeDtypeStruct((NC * N, 1), jnp.float32),
                  jax.ShapeDtypeStruct((NC * N, 1), jnp.float32)),
        mesh=_sc_mesh(),
        compiler_params=_SC_PARAMS,
        scratch_types=[
            pltpu.VMEM((CHUNK,), jnp.int32),
            pltpu.VMEM((CHUNK, 1), jnp.float32),
            pltpu.VMEM_SHARED((N, 1), jnp.float32),
            pltpu.VMEM_SHARED((N, 1), jnp.float32),
        ],
    )
    def deg_kernel(src_hbm, dst_hbm, ones_hbm, zeros_hbm, dego_hbm, degi_hbm,
                   idx_v, ones_v, dego_s, degi_s):
        cid = lax.axis_index("c")
        sid = lax.axis_index("s")
        pltpu.sync_copy(ones_hbm, ones_v)

        @pl.when(sid < nsplit)
        def _init():
            r0 = sid * rows_per
            pltpu.sync_copy(zeros_hbm.at[pl.ds(r0, rows_per)],
                            dego_s.at[pl.ds(r0, rows_per)])
            pltpu.sync_copy(zeros_hbm.at[pl.ds(r0, rows_per)],
                            degi_s.at[pl.ds(r0, rows_per)])

        plsc.subcore_barrier()
        base = (cid * NS + sid) * per_tile
        for j in range(nchunk):
            off = base + j * CHUNK
            pltpu.sync_copy(src_hbm.at[pl.ds(off, CHUNK)], idx_v)
            pltpu.sync_copy(ones_v, dego_s.at[idx_v], add=True)
            pltpu.sync_copy(dst_hbm.at[pl.ds(off, CHUNK)], idx_v)
            pltpu.sync_copy(ones_v, degi_s.at[idx_v], add=True)
        plsc.subcore_barrier()

        @pl.when(sid < nsplit)
        def _writeout():
            r0 = sid * rows_per
            pltpu.sync_copy(dego_s.at[pl.ds(r0, rows_per)],
                            dego_hbm.at[pl.ds(cid * N + r0, rows_per)])
            pltpu.sync_copy(degi_s.at[pl.ds(r0, rows_per)],
                            degi_hbm.at[pl.ds(cid * N + r0, rows_per)])

    return deg_kernel


def _make_agg_kernel(E, N, H):
    per_tile = E // NW
    nchunk = per_tile // CHUNK
    rows_per = N // NS

    @functools.partial(
        pl.kernel,
        out_type=jax.ShapeDtypeStruct((NC * N, H), jnp.float32),
        mesh=_sc_mesh(),
        compiler_params=_SC_PARAMS,
        scratch_types=[
            pltpu.VMEM((CHUNK,), jnp.int32),
            pltpu.VMEM((CHUNK,), jnp.int32),
            pltpu.VMEM((CHUNK, H), jnp.float32),
            pltpu.SemaphoreType.DMA,
            pltpu.VMEM_SHARED((N, H), jnp.float32),
            pltpu.VMEM_SHARED((N, H), jnp.float32),
        ],
    )
    def agg_kernel(g_hbm, src_hbm, dst_hbm, zeros_hbm, out_hbm,
                   sidx_v, didx_v, rows_v, sem, g_s, agg_s):
        cid = lax.axis_index("c")
        sid = lax.axis_index("s")
        r0 = sid * rows_per
        pltpu.sync_copy(g_hbm.at[pl.ds(r0, rows_per)],
                        g_s.at[pl.ds(r0, rows_per)])
        pltpu.sync_copy(zeros_hbm.at[pl.ds(r0, rows_per)],
                        agg_s.at[pl.ds(r0, rows_per)])
        plsc.subcore_barrier()
        base = (cid * NS + sid) * per_tile
        for j in range(nchunk):
            off = base + j * CHUNK
            pltpu.sync_copy(src_hbm.at[pl.ds(off, CHUNK)], sidx_v)
            pltpu.sync_copy(dst_hbm.at[pl.ds(off, CHUNK)], didx_v)
            pltpu.async_copy(g_s.at[sidx_v], rows_v, sem).wait()
            pltpu.sync_copy(rows_v, agg_s.at[didx_v], add=True)
        plsc.subcore_barrier()
        pltpu.sync_copy(agg_s.at[pl.ds(r0, rows_per)],
                        out_hbm.at[pl.ds(cid * N + r0, rows_per)])

    return agg_kernel


def _head_body(x_ref, w_ref, b_ref, dego_ref, degi_ref,
               g_ref, ns_ref, nd_ref):
    h = jnp.dot(x_ref[...], w_ref[...], preferred_element_type=jnp.float32)
    h = jnp.maximum(h + b_ref[...], 0.0)
    ns = lax.rsqrt(dego_ref[0] + dego_ref[1] + 1.0)
    nd = lax.rsqrt(degi_ref[0] + degi_ref[1] + 1.0)
    g_ref[...] = h * ns
    ns_ref[...] = ns
    nd_ref[...] = nd


def _mid_body(parts_ref, g_ref, ns_ref, nd_ref, w_ref, b_ref, out_ref):
    agg = (parts_ref[0] + parts_ref[1] + g_ref[...]) * nd_ref[...]
    h = jnp.dot(agg, w_ref[...], preferred_element_type=jnp.float32)
    h = jnp.maximum(h + b_ref[...], 0.0)
    out_ref[...] = h * ns_ref[...]


def _tail_body(parts_ref, g_ref, nd_ref, wc_ref, bc_ref, w2_ref, b2_ref,
               out_ref):
    agg = (parts_ref[0] + parts_ref[1] + g_ref[...]) * nd_ref[...]
    h = jnp.dot(agg, wc_ref[...], preferred_element_type=jnp.float32)
    h = jnp.maximum(h + bc_ref[...], 0.0)
    out_ref[...] = (jnp.dot(h, w2_ref[...],
                            preferred_element_type=jnp.float32)
                    + b2_ref[...])


def kernel(x, edge_index, W1, b1, Wc1, bc1, Wc2, bc2, W2, b2):
    N, F = x.shape
    H = W1.shape[1]
    C = W2.shape[1]
    E = edge_index.shape[1]
    src = edge_index[0]
    dst = edge_index[1]
    ones_c = jnp.ones((CHUNK, 1), jnp.float32)
    zeros_n1 = jnp.zeros((N, 1), jnp.float32)
    zeros_nh = jnp.zeros((N, H), jnp.float32)

    deg_k = _make_degree_kernel(E, N)
    agg_k = _make_agg_kernel(E, N, H)

    dego_p, degi_p = deg_k(src, dst, ones_c, zeros_n1)
    dego_p = dego_p.reshape(NC, N, 1)
    degi_p = degi_p.reshape(NC, N, 1)

    g0, ns, nd = pl.pallas_call(
        _head_body,
        out_shape=(jax.ShapeDtypeStruct((N, H), jnp.float32),
                   jax.ShapeDtypeStruct((N, 1), jnp.float32),
                   jax.ShapeDtypeStruct((N, 1), jnp.float32)),
    )(x, W1, b1.reshape(1, H), dego_p, degi_p)

    agg1 = agg_k(g0, src, dst, zeros_nh).reshape(NC, N, H)
    g1 = pl.pallas_call(
        _mid_body,
        out_shape=jax.ShapeDtypeStruct((N, H), jnp.float32),
    )(agg1, g0, ns, nd, Wc1, bc1.reshape(1, H))

    agg2 = agg_k(g1, src, dst, zeros_nh).reshape(NC, N, H)
    out = pl.pallas_call(
        _tail_body,
        out_shape=jax.ShapeDtypeStruct((N, C), jnp.float32),
    )(agg2, g1, nd, Wc2, bc2.reshape(1, H), W2, b2.reshape(1, C))
    return out


# speedup vs baseline: 38.3934x; 1.4279x over previous
import functools

import jax
import jax.numpy as jnp
from jax import lax
from jax.experimental import pallas as pl
from jax.experimental.pallas import tpu as pltpu
from jax.experimental.pallas import tpu_sc as plsc

NC, NS = 2, 16
NW = NC * NS
CHUNK = 2000


def _sc_mesh():
    return plsc.VectorSubcoreMesh(core_axis_name="c", subcore_axis_name="s",
                                  num_cores=NC, num_subcores=NS)


_SC_PARAMS = pltpu.CompilerParams(use_tc_tiling_on_sc=False)


def _make_degree_kernel(E, N, H):
    per_tile = E // NW
    nchunk = per_tile // CHUNK
    assert per_tile % CHUNK == 0 and E % NW == 0
    rows_per = N // NS
    out_sds = jax.ShapeDtypeStruct((N, H), jnp.float32)

    @functools.partial(
        pl.kernel,
        out_type=(out_sds, out_sds, out_sds, out_sds),
        mesh=_sc_mesh(),
        compiler_params=_SC_PARAMS,
        scratch_types=[
            pltpu.VMEM((CHUNK,), jnp.int32),
            pltpu.VMEM((CHUNK, H), jnp.float32),
            pltpu.VMEM_SHARED((N, H), jnp.float32),
            pltpu.VMEM_SHARED((N, H), jnp.float32),
        ],
    )
    def deg_kernel(src_hbm, dst_hbm, ones_hbm, zeros_hbm,
                   dego0_hbm, dego1_hbm, degi0_hbm, degi1_hbm,
                   idx_v, ones_v, dego_s, degi_s):
        cid = lax.axis_index("c")
        sid = lax.axis_index("s")
        r0 = sid * rows_per
        pltpu.sync_copy(ones_hbm, ones_v)
        pltpu.sync_copy(zeros_hbm.at[pl.ds(r0, rows_per)],
                        dego_s.at[pl.ds(r0, rows_per)])
        pltpu.sync_copy(zeros_hbm.at[pl.ds(r0, rows_per)],
                        degi_s.at[pl.ds(r0, rows_per)])
        plsc.subcore_barrier()
        base = (cid * NS + sid) * per_tile
        for j in range(nchunk):
            off = base + j * CHUNK
            pltpu.sync_copy(src_hbm.at[pl.ds(off, CHUNK)], idx_v)
            pltpu.sync_copy(ones_v, dego_s.at[idx_v], add=True)
            pltpu.sync_copy(dst_hbm.at[pl.ds(off, CHUNK)], idx_v)
            pltpu.sync_copy(ones_v, degi_s.at[idx_v], add=True)
        plsc.subcore_barrier()

        @pl.when(cid == 0)
        def _wr0():
            pltpu.sync_copy(dego_s.at[pl.ds(r0, rows_per)],
                            dego0_hbm.at[pl.ds(r0, rows_per)])
            pltpu.sync_copy(degi_s.at[pl.ds(r0, rows_per)],
                            degi0_hbm.at[pl.ds(r0, rows_per)])

        @pl.when(cid == 1)
        def _wr1():
            pltpu.sync_copy(dego_s.at[pl.ds(r0, rows_per)],
                            dego1_hbm.at[pl.ds(r0, rows_per)])
            pltpu.sync_copy(degi_s.at[pl.ds(r0, rows_per)],
                            degi1_hbm.at[pl.ds(r0, rows_per)])

    return deg_kernel


def _make_agg_kernel(E, N, H):
    per_tile = E // NW
    nchunk = per_tile // CHUNK
    rows_per = N // NS
    out_sds = jax.ShapeDtypeStruct((N, H), jnp.float32)

    @functools.partial(
        pl.kernel,
        out_type=(out_sds, out_sds),
        mesh=_sc_mesh(),
        compiler_params=_SC_PARAMS,
        scratch_types=[
            pltpu.VMEM((CHUNK,), jnp.int32),
            pltpu.VMEM((CHUNK,), jnp.int32),
            pltpu.VMEM((CHUNK, H), jnp.float32),
            pltpu.SemaphoreType.DMA,
            pltpu.VMEM_SHARED((N, H), jnp.float32),
            pltpu.VMEM_SHARED((N, H), jnp.float32),
        ],
    )
    def agg_kernel(g_hbm, src_hbm, dst_hbm, zeros_hbm, out0_hbm, out1_hbm,
                   sidx_v, didx_v, rows_v, sem, g_s, agg_s):
        cid = lax.axis_index("c")
        sid = lax.axis_index("s")
        r0 = sid * rows_per
        pltpu.sync_copy(g_hbm.at[pl.ds(r0, rows_per)],
                        g_s.at[pl.ds(r0, rows_per)])
        pltpu.sync_copy(zeros_hbm.at[pl.ds(r0, rows_per)],
                        agg_s.at[pl.ds(r0, rows_per)])
        plsc.subcore_barrier()
        base = (cid * NS + sid) * per_tile
        for j in range(nchunk):
            off = base + j * CHUNK
            pltpu.sync_copy(src_hbm.at[pl.ds(off, CHUNK)], sidx_v)
            pltpu.sync_copy(dst_hbm.at[pl.ds(off, CHUNK)], didx_v)
            pltpu.async_copy(g_s.at[sidx_v], rows_v, sem).wait()
            pltpu.sync_copy(rows_v, agg_s.at[didx_v], add=True)
        plsc.subcore_barrier()

        @pl.when(cid == 0)
        def _wr0():
            pltpu.sync_copy(agg_s.at[pl.ds(r0, rows_per)],
                            out0_hbm.at[pl.ds(r0, rows_per)])

        @pl.when(cid == 1)
        def _wr1():
            pltpu.sync_copy(agg_s.at[pl.ds(r0, rows_per)],
                            out1_hbm.at[pl.ds(r0, rows_per)])

    return agg_kernel


def _head_body(x_ref, w_ref, b_ref, dego0_ref, dego1_ref, degi0_ref,
               degi1_ref, g_ref, ns_ref, nd_ref):
    h = jnp.dot(x_ref[...], w_ref[...], preferred_element_type=jnp.float32)
    hp = jnp.maximum(h + b_ref[...], 0.0)
    ns = lax.rsqrt(dego0_ref[...] + dego1_ref[...] + 1.0)
    nd = lax.rsqrt(degi0_ref[...] + degi1_ref[...] + 1.0)
    g_ref[...] = hp * ns
    ns_ref[...] = ns
    nd_ref[...] = nd


def _mid_body(p0_ref, p1_ref, g_ref, ns_ref, nd_ref, w_ref, b_ref, out_ref):
    agg = (p0_ref[...] + p1_ref[...] + g_ref[...]) * nd_ref[...]
    h = jnp.dot(agg, w_ref[...], preferred_element_type=jnp.float32)
    h = jnp.maximum(h + b_ref[...], 0.0)
    out_ref[...] = h * ns_ref[...]


def _tail_body(p0_ref, p1_ref, g_ref, nd_ref, wc_ref, bc_ref, w2_ref, b2_ref,
               out_ref):
    agg = (p0_ref[...] + p1_ref[...] + g_ref[...]) * nd_ref[...]
    h = jnp.dot(agg, wc_ref[...], preferred_element_type=jnp.float32)
    h = jnp.maximum(h + bc_ref[...], 0.0)
    out_ref[...] = (jnp.dot(h, w2_ref[...],
                            preferred_element_type=jnp.float32)
                    + b2_ref[...])


def kernel(x, edge_index, W1, b1, Wc1, bc1, Wc2, bc2, W2, b2):
    N, F = x.shape
    H = W1.shape[1]
    C = W2.shape[1]
    E = edge_index.shape[1]
    src = edge_index[0]
    dst = edge_index[1]
    P = N * H // 128
    ones_c = jnp.ones((CHUNK, H), jnp.float32)
    zeros_nh = jnp.zeros((N, H), jnp.float32)
    eye8 = jnp.eye(8, dtype=jnp.float32)
    w1_blk = jnp.kron(eye8, W1)
    wc1_blk = jnp.kron(eye8, Wc1)
    wc2_blk = jnp.kron(eye8, Wc2)
    w2_blk = jnp.kron(eye8, W2)
    b1_t = jnp.tile(b1, 8).reshape(1, 128)
    bc1_t = jnp.tile(bc1, 8).reshape(1, 128)
    bc2_t = jnp.tile(bc2, 8).reshape(1, 128)
    b2_t = jnp.tile(b2, 8).reshape(1, 8 * C)
    x_p = x.reshape(N // 8, 8 * F)

    deg_k = _make_degree_kernel(E, N, H)
    agg_k = _make_agg_kernel(E, N, H)

    dego0, dego1, degi0, degi1 = deg_k(src, dst, ones_c, zeros_nh)
    pk = (P, 128)
    g0p, nsp, ndp = pl.pallas_call(
        _head_body,
        out_shape=(jax.ShapeDtypeStruct(pk, jnp.float32),) * 3,
    )(x_p, w1_blk, b1_t, dego0.reshape(pk), dego1.reshape(pk),
      degi0.reshape(pk), degi1.reshape(pk))

    p0, p1 = agg_k(g0p.reshape(N, H), src, dst, zeros_nh)
    g1p = pl.pallas_call(
        _mid_body,
        out_shape=jax.ShapeDtypeStruct(pk, jnp.float32),
    )(p0.reshape(pk), p1.reshape(pk), g0p, nsp, ndp, wc1_blk, bc1_t)

    q0, q1 = agg_k(g1p.reshape(N, H), src, dst, zeros_nh)
    out_p = pl.pallas_call(
        _tail_body,
        out_shape=jax.ShapeDtypeStruct((P, 8 * C), jnp.float32),
    )(q0.reshape(pk), q1.reshape(pk), g1p, ndp, wc2_blk, bc2_t, w2_blk,
      b2_t)
    return out_p.reshape(N, C)


# speedup vs baseline: 43.9388x; 1.1444x over previous
import functools

import jax
import jax.numpy as jnp
from jax import lax
from jax.experimental import pallas as pl
from jax.experimental.pallas import tpu as pltpu
from jax.experimental.pallas import tpu_sc as plsc

NC, NS = 2, 16
NW = NC * NS
CHUNK = 2000


def _sc_mesh():
    return plsc.VectorSubcoreMesh(core_axis_name="c", subcore_axis_name="s",
                                  num_cores=NC, num_subcores=NS)


_SC_PARAMS = pltpu.CompilerParams(use_tc_tiling_on_sc=False)


def _make_degree_kernel(E, N, H):
    per_tile = E // NW
    nchunk = per_tile // CHUNK
    assert per_tile % CHUNK == 0 and E % NW == 0
    rows_per = N // NS
    out_sds = jax.ShapeDtypeStruct((N, H), jnp.float32)

    @functools.partial(
        pl.kernel,
        out_type=(out_sds, out_sds, out_sds, out_sds),
        mesh=_sc_mesh(),
        compiler_params=_SC_PARAMS,
        scratch_types=[
            pltpu.VMEM((CHUNK,), jnp.int32),
            pltpu.VMEM((CHUNK,), jnp.int32),
            pltpu.VMEM((CHUNK,), jnp.int32),
            pltpu.VMEM((CHUNK,), jnp.int32),
            pltpu.VMEM((CHUNK, H), jnp.float32),
            pltpu.SemaphoreType.DMA,
            pltpu.SemaphoreType.DMA,
            pltpu.SemaphoreType.DMA,
            pltpu.SemaphoreType.DMA,
            pltpu.SemaphoreType.DMA,
            pltpu.SemaphoreType.DMA,
            pltpu.SemaphoreType.DMA,
            pltpu.SemaphoreType.DMA,
            pltpu.VMEM_SHARED((N, H), jnp.float32),
            pltpu.VMEM_SHARED((N, H), jnp.float32),
        ],
    )
    def deg_kernel(src_hbm, dst_hbm, ones_hbm, zeros_hbm,
                   dego0_hbm, dego1_hbm, degi0_hbm, degi1_hbm,
                   ib0, ib1, ib2, ib3, ones_v,
                   si0, si1, si2, si3, ss0, ss1, ss2, ss3,
                   dego_s, degi_s):
        cid = lax.axis_index("c")
        sid = lax.axis_index("s")
        r0 = sid * rows_per
        base = (cid * NS + sid) * per_tile
        ib = [ib0, ib1, ib2, ib3]
        sem_i = [si0, si1, si2, si3]
        sem_s = [ss0, ss1, ss2, ss3]
        ntask = 2 * nchunk

        def idx_load(k):
            off = base + (k // 2) * CHUNK
            hbm = src_hbm if k % 2 == 0 else dst_hbm
            return pltpu.async_copy(hbm.at[pl.ds(off, CHUNK)], ib[k % 4],
                                    sem_i[k % 4])

        idx_d = {k: idx_load(k) for k in range(min(2, ntask))}
        pltpu.sync_copy(ones_hbm, ones_v)
        pltpu.sync_copy(zeros_hbm.at[pl.ds(r0, rows_per)],
                        dego_s.at[pl.ds(r0, rows_per)])
        pltpu.sync_copy(zeros_hbm.at[pl.ds(r0, rows_per)],
                        degi_s.at[pl.ds(r0, rows_per)])
        plsc.subcore_barrier()
        sc_d = {}
        for k in range(ntask):
            idx_d[k].wait()
            tbl = dego_s if k % 2 == 0 else degi_s
            sc_d[k] = pltpu.async_copy(ones_v, tbl.at[ib[k % 4]],
                                       sem_s[k % 4], add=True)
            if k + 2 < ntask:
                if k - 2 >= 0:
                    sc_d[k - 2].wait()
                idx_d[k + 2] = idx_load(k + 2)
        for k in range(max(0, ntask - 4), ntask):
            sc_d[k].wait()
        plsc.subcore_barrier()

        @pl.when(cid == 0)
        def _wr0():
            pltpu.sync_copy(dego_s.at[pl.ds(r0, rows_per)],
                            dego0_hbm.at[pl.ds(r0, rows_per)])
            pltpu.sync_copy(degi_s.at[pl.ds(r0, rows_per)],
                            degi0_hbm.at[pl.ds(r0, rows_per)])

        @pl.when(cid == 1)
        def _wr1():
            pltpu.sync_copy(dego_s.at[pl.ds(r0, rows_per)],
                            dego1_hbm.at[pl.ds(r0, rows_per)])
            pltpu.sync_copy(degi_s.at[pl.ds(r0, rows_per)],
                            degi1_hbm.at[pl.ds(r0, rows_per)])

    return deg_kernel


def _make_agg_kernel(E, N, H):
    per_tile = E // NW
    nchunk = per_tile // CHUNK
    rows_per = N // NS
    out_sds = jax.ShapeDtypeStruct((N, H), jnp.float32)

    @functools.partial(
        pl.kernel,
        out_type=(out_sds, out_sds),
        mesh=_sc_mesh(),
        compiler_params=_SC_PARAMS,
        scratch_types=[
            pltpu.VMEM((CHUNK,), jnp.int32),
            pltpu.VMEM((CHUNK,), jnp.int32),
            pltpu.VMEM((CHUNK,), jnp.int32),
            pltpu.VMEM((CHUNK,), jnp.int32),
            pltpu.VMEM((CHUNK, H), jnp.float32),
            pltpu.VMEM((CHUNK, H), jnp.float32),
            pltpu.SemaphoreType.DMA,
            pltpu.SemaphoreType.DMA,
            pltpu.SemaphoreType.DMA,
            pltpu.SemaphoreType.DMA,
            pltpu.SemaphoreType.DMA,
            pltpu.SemaphoreType.DMA,
            pltpu.SemaphoreType.DMA,
            pltpu.SemaphoreType.DMA,
            pltpu.VMEM_SHARED((N, H), jnp.float32),
            pltpu.VMEM_SHARED((N, H), jnp.float32),
        ],
    )
    def agg_kernel(g_hbm, src_hbm, dst_hbm, zeros_hbm, out0_hbm, out1_hbm,
                   sx0, sx1, dx0, dx1, rw0, rw1,
                   ssi0, ssi1, sdi0, sdi1, sg0, sg1, ssc0, ssc1,
                   g_s, agg_s):
        cid = lax.axis_index("c")
        sid = lax.axis_index("s")
        r0 = sid * rows_per
        base = (cid * NS + sid) * per_tile
        sx = [sx0, sx1]
        dx = [dx0, dx1]
        rw = [rw0, rw1]
        sem_si = [ssi0, ssi1]
        sem_di = [sdi0, sdi1]
        sem_g = [sg0, sg1]
        sem_s = [ssc0, ssc1]

        def idx_load(j):
            off = base + j * CHUNK
            b = j % 2
            return (pltpu.async_copy(src_hbm.at[pl.ds(off, CHUNK)], sx[b],
                                     sem_si[b]),
                    pltpu.async_copy(dst_hbm.at[pl.ds(off, CHUNK)], dx[b],
                                     sem_di[b]))

        idx_d = {0: idx_load(0)}
        pltpu.sync_copy(g_hbm.at[pl.ds(r0, rows_per)],
                        g_s.at[pl.ds(r0, rows_per)])
        pltpu.sync_copy(zeros_hbm.at[pl.ds(r0, rows_per)],
                        agg_s.at[pl.ds(r0, rows_per)])
        plsc.subcore_barrier()
        idx_d[0][0].wait()
        idx_d[0][1].wait()
        gat_d = {0: pltpu.async_copy(g_s.at[sx[0]], rw[0], sem_g[0])}
        sca_d = {}
        for j in range(nchunk):
            b = j % 2
            gat_d[j].wait()
            sca_d[j] = pltpu.async_copy(rw[b], agg_s.at[dx[b]], sem_s[b],
                                        add=True)
            if j + 1 < nchunk:
                nb = (j + 1) % 2
                if j - 1 >= 0:
                    sca_d[j - 1].wait()
                idx_d[j + 1] = idx_load(j + 1)
                idx_d[j + 1][0].wait()
                idx_d[j + 1][1].wait()
                gat_d[j + 1] = pltpu.async_copy(g_s.at[sx[nb]], rw[nb],
                                                sem_g[nb])
        for j in range(max(0, nchunk - 2), nchunk):
            sca_d[j].wait()
        plsc.subcore_barrier()

        @pl.when(cid == 0)
        def _wr0():
            pltpu.sync_copy(agg_s.at[pl.ds(r0, rows_per)],
                            out0_hbm.at[pl.ds(r0, rows_per)])

        @pl.when(cid == 1)
        def _wr1():
            pltpu.sync_copy(agg_s.at[pl.ds(r0, rows_per)],
                            out1_hbm.at[pl.ds(r0, rows_per)])

    return agg_kernel


def _head_body(x_ref, w_ref, b_ref, dego0_ref, dego1_ref, degi0_ref,
               degi1_ref, g_ref, ns_ref, nd_ref):
    h = jnp.dot(x_ref[...], w_ref[...], preferred_element_type=jnp.float32)
    hp = jnp.maximum(h + b_ref[...], 0.0)
    ns = lax.rsqrt(dego0_ref[...] + dego1_ref[...] + 1.0)
    nd = lax.rsqrt(degi0_ref[...] + degi1_ref[...] + 1.0)
    g_ref[...] = hp * ns
    ns_ref[...] = ns
    nd_ref[...] = nd


def _mid_body(p0_ref, p1_ref, g_ref, ns_ref, nd_ref, w_ref, b_ref, out_ref):
    agg = (p0_ref[...] + p1_ref[...] + g_ref[...]) * nd_ref[...]
    h = jnp.dot(agg, w_ref[...], preferred_element_type=jnp.float32)
    h = jnp.maximum(h + b_ref[...], 0.0)
    out_ref[...] = h * ns_ref[...]


def _tail_body(p0_ref, p1_ref, g_ref, nd_ref, wc_ref, bc_ref, w2_ref, b2_ref,
               out_ref):
    agg = (p0_ref[...] + p1_ref[...] + g_ref[...]) * nd_ref[...]
    h = jnp.dot(agg, wc_ref[...], preferred_element_type=jnp.float32)
    h = jnp.maximum(h + bc_ref[...], 0.0)
    out_ref[...] = (jnp.dot(h, w2_ref[...],
                            preferred_element_type=jnp.float32)
                    + b2_ref[...])


def kernel(x, edge_index, W1, b1, Wc1, bc1, Wc2, bc2, W2, b2):
    N, F = x.shape
    H = W1.shape[1]
    C = W2.shape[1]
    E = edge_index.shape[1]
    src = edge_index[0]
    dst = edge_index[1]
    P = N * H // 128
    ones_c = jnp.ones((CHUNK, H), jnp.float32)
    zeros_nh = jnp.zeros((N, H), jnp.float32)
    eye8 = jnp.eye(8, dtype=jnp.float32)
    w1_blk = jnp.kron(eye8, W1)
    wc1_blk = jnp.kron(eye8, Wc1)
    wc2_blk = jnp.kron(eye8, Wc2)
    w2_blk = jnp.kron(eye8, W2)
    b1_t = jnp.tile(b1, 8).reshape(1, 128)
    bc1_t = jnp.tile(bc1, 8).reshape(1, 128)
    bc2_t = jnp.tile(bc2, 8).reshape(1, 128)
    b2_t = jnp.tile(b2, 8).reshape(1, 8 * C)
    x_p = x.reshape(N // 8, 8 * F)

    deg_k = _make_degree_kernel(E, N, H)
    agg_k = _make_agg_kernel(E, N, H)

    dego0, dego1, degi0, degi1 = deg_k(src, dst, ones_c, zeros_nh)
    pk = (P, 128)
    g0p, nsp, ndp = pl.pallas_call(
        _head_body,
        out_shape=(jax.ShapeDtypeStruct(pk, jnp.float32),) * 3,
    )(x_p, w1_blk, b1_t, dego0.reshape(pk), dego1.reshape(pk),
      degi0.reshape(pk), degi1.reshape(pk))

    p0, p1 = agg_k(g0p.reshape(N, H), src, dst, zeros_nh)
    g1p = pl.pallas_call(
        _mid_body,
        out_shape=jax.ShapeDtypeStruct(pk, jnp.float32),
    )(p0.reshape(pk), p1.reshape(pk), g0p, nsp, ndp, wc1_blk, bc1_t)

    q0, q1 = agg_k(g1p.reshape(N, H), src, dst, zeros_nh)
    out_p = pl.pallas_call(
        _tail_body,
        out_shape=jax.ShapeDtypeStruct((P, 8 * C), jnp.float32),
    )(q0.reshape(pk), q1.reshape(pk), g1p, ndp, wc2_blk, bc2_t, w2_blk,
      b2_t)
    return out_p.reshape(N, C)
